# Initial kernel scaffold; baseline (speedup 1.0000x reference)
import jax
import jax.numpy as jnp
from jax.experimental import pallas as pl


def kernel(year, month, day, weekday, stores, items, emb_month, emb_day, emb_weekday, emb_stores, emb_items, W1, b1, W2, b2, W3, b3):
    raise NotImplementedError("write your pallas kernel here")


# speedup vs baseline: 4.7850x; 4.7850x over previous
import jax
import jax.numpy as jnp
from jax import lax
from jax.experimental import pallas as pl
from jax.experimental.pallas import tpu as pltpu

_OFF_M, _OFF_D, _OFF_W, _OFF_S, _OFF_I, _COL_Y = 0, 13, 45, 53, 64, 115


def _fused_body(year_ref, month_ref, day_ref, weekday_ref, stores_ref,
                items_ref, tbl_ref, w1_ref, b1_ref, w2_ref, b2_ref,
                w3_ref, b3_ref, out_ref):
    bblk = year_ref.shape[0]
    col = lax.broadcasted_iota(jnp.int32, (bblk, 128), 1)
    hot = ((col == month_ref[...] + _OFF_M)
           | (col == day_ref[...] + _OFF_D)
           | (col == weekday_ref[...] + _OFF_W)
           | (col == stores_ref[...] + _OFF_S)
           | (col == items_ref[...] + _OFF_I))
    oh = jnp.where(col == _COL_Y, year_ref[...],
                   hot.astype(jnp.float32))
    hc = jnp.dot(oh, tbl_ref[...], preferred_element_type=jnp.float32)
    h1 = jnp.dot(hc, w1_ref[...], preferred_element_type=jnp.float32)
    h1 = jnp.maximum(h1 + b1_ref[...], 0.0)
    h2 = jnp.dot(h1, w2_ref[...], preferred_element_type=jnp.float32)
    h2 = jnp.maximum(h2 + b2_ref[...], 0.0)
    out_ref[...] = (jnp.dot(h2, w3_ref[...],
                            preferred_element_type=jnp.float32)
                    + b3_ref[...])


def kernel(year, month, day, weekday, stores, items, emb_month, emb_day,
           emb_weekday, emb_stores, emb_items, W1, b1, W2, b2, W3, b3):
    B = year.shape[0]
    bblk = 4096
    grid = (B // bblk,)

    tbl = jnp.zeros((128, 64), jnp.float32)
    tbl = tbl.at[_OFF_M:_OFF_M + 13, 1:8].set(emb_month)
    tbl = tbl.at[_OFF_D:_OFF_D + 32, 8:24].set(emb_day)
    tbl = tbl.at[_OFF_W:_OFF_W + 8, 24:28].set(emb_weekday)
    tbl = tbl.at[_OFF_S:_OFF_S + 11, 28:34].set(emb_stores)
    tbl = tbl.at[_OFF_I:_OFF_I + 51, 34:60].set(emb_items)
    tbl = tbl.at[_COL_Y, 0].set(1.0)
    w1p = jnp.zeros((64, 100), jnp.float32).at[:60].set(W1)

    blk = pl.BlockSpec((bblk, 1), lambda i: (i, 0))
    full = lambda s: pl.BlockSpec(s, lambda i: (0,) * len(s))
    return pl.pallas_call(
        _fused_body,
        grid=grid,
        in_specs=[blk] * 6 + [full((128, 64)), full((64, 100)),
                              full((1, 100)), full((100, 10)),
                              full((1, 10)), full((10, 1)), full((1, 1))],
        out_specs=pl.BlockSpec((bblk, 1), lambda i: (i, 0)),
        out_shape=jax.ShapeDtypeStruct((B, 1), jnp.float32),
        compiler_params=pltpu.CompilerParams(
            dimension_semantics=("arbitrary",)),
    )(year, month, day, weekday, stores, items, tbl, w1p,
      b1.reshape(1, 100), W2, b2.reshape(1, 10), W3, b3.reshape(1, 1))


# speedup vs baseline: 5.0681x; 1.0592x over previous
import jax
import jax.numpy as jnp
from jax import lax
from jax.experimental import pallas as pl
from jax.experimental.pallas import tpu as pltpu

_OFF_M, _OFF_D, _OFF_W, _OFF_S, _OFF_I, _COL_Y = 0, 13, 45, 53, 64, 115


def _fused_body(year_ref, month_ref, day_ref, weekday_ref, stores_ref,
                items_ref, emb_month_ref, emb_day_ref, emb_weekday_ref,
                emb_stores_ref, emb_items_ref, w1_ref, b1_ref, w2_ref,
                b2_ref, w3_ref, b3_ref, out_ref, tbl_ref, w1p_ref):
    i = pl.program_id(0)

    @pl.when(i == 0)
    def _assemble():
        tbl_ref[...] = jnp.zeros_like(tbl_ref)
        tbl_ref[_OFF_M:_OFF_M + 13, 1:8] = emb_month_ref[...]
        tbl_ref[_OFF_D:_OFF_D + 32, 8:24] = emb_day_ref[...]
        tbl_ref[_OFF_W:_OFF_W + 8, 24:28] = emb_weekday_ref[...]
        tbl_ref[_OFF_S:_OFF_S + 11, 28:34] = emb_stores_ref[...]
        tbl_ref[_OFF_I:_OFF_I + 51, 34:60] = emb_items_ref[...]
        tbl_ref[_COL_Y:_COL_Y + 1, 0:1] = jnp.ones((1, 1), jnp.float32)
        w1p_ref[...] = jnp.zeros_like(w1p_ref)
        w1p_ref[0:60, :] = w1_ref[...]

    bblk = year_ref.shape[0]
    col = lax.broadcasted_iota(jnp.int32, (bblk, 128), 1)
    hot = ((col == month_ref[...] + _OFF_M)
           | (col == day_ref[...] + _OFF_D)
           | (col == weekday_ref[...] + _OFF_W)
           | (col == stores_ref[...] + _OFF_S)
           | (col == items_ref[...] + _OFF_I))
    oh = jnp.where(col == _COL_Y, year_ref[...],
                   hot.astype(jnp.float32))
    hc = jnp.dot(oh, tbl_ref[...], preferred_element_type=jnp.float32)
    h1 = jnp.dot(hc, w1p_ref[...], preferred_element_type=jnp.float32)
    h1 = jnp.maximum(h1 + b1_ref[...], 0.0)
    h2 = jnp.dot(h1, w2_ref[...], preferred_element_type=jnp.float32)
    h2 = jnp.maximum(h2 + b2_ref[...], 0.0)
    out_ref[...] = (jnp.dot(h2, w3_ref[...],
                            preferred_element_type=jnp.float32)
                    + b3_ref[...])


def kernel(year, month, day, weekday, stores, items, emb_month, emb_day,
           emb_weekday, emb_stores, emb_items, W1, b1, W2, b2, W3, b3):
    B = year.shape[0]
    bblk = 4096
    grid = (B // bblk,)

    blk = pl.BlockSpec((bblk, 1), lambda i: (i, 0))
    full = lambda s: pl.BlockSpec(s, lambda i: (0,) * len(s))
    return pl.pallas_call(
        _fused_body,
        grid=grid,
        in_specs=[blk] * 6 + [full(emb_month.shape), full(emb_day.shape),
                              full(emb_weekday.shape),
                              full(emb_stores.shape), full(emb_items.shape),
                              full(W1.shape), full(b1.shape), full(W2.shape),
                              full(b2.shape), full(W3.shape), full(b3.shape)],
        out_specs=pl.BlockSpec((bblk, 1), lambda i: (i, 0)),
        out_shape=jax.ShapeDtypeStruct((B, 1), jnp.float32),
        scratch_shapes=[pltpu.VMEM((128, 64), jnp.float32),
                        pltpu.VMEM((64, 100), jnp.float32)],
        compiler_params=pltpu.CompilerParams(
            dimension_semantics=("arbitrary",)),
    )(year, month, day, weekday, stores, items, emb_month, emb_day,
      emb_weekday, emb_stores, emb_items, W1, b1, W2, b2, W3, b3)


# speedup vs baseline: 24.0851x; 4.7523x over previous
import jax
import jax.numpy as jnp
from jax import lax
from jax.experimental import pallas as pl
from jax.experimental.pallas import tpu as pltpu

_OFF_M, _OFF_D, _OFF_W, _OFF_S, _OFF_I, _ROW_Y, _ROW_1 = (
    0, 13, 45, 53, 64, 115, 116)


def _dgT(a, b):
    return lax.dot_general(a, b, (((0,), (0,)), ((), ())),
                           preferred_element_type=jnp.float32)


def _fused_body(year_ref, month_ref, day_ref, weekday_ref, stores_ref,
                items_ref, emb_month_ref, emb_day_ref, emb_weekday_ref,
                emb_stores_ref, emb_items_ref, w1_ref, b1_ref, w2_ref,
                b2_ref, w3_ref, b3_ref, out_ref, tbl_ref, w1p_ref,
                w2e_ref, w3e_ref):
    i = pl.program_id(0)

    @pl.when(i == 0)
    def _assemble():
        tbl_ref[...] = jnp.zeros_like(tbl_ref)
        tbl_ref[_OFF_M:_OFF_M + 13, 1:8] = emb_month_ref[...]
        tbl_ref[_OFF_D:_OFF_D + 32, 8:24] = emb_day_ref[...]
        tbl_ref[_OFF_W:_OFF_W + 8, 24:28] = emb_weekday_ref[...]
        tbl_ref[_OFF_S:_OFF_S + 11, 28:34] = emb_stores_ref[...]
        tbl_ref[_OFF_I:_OFF_I + 51, 34:60] = emb_items_ref[...]
        tbl_ref[_ROW_Y:_ROW_Y + 1, 0:1] = jnp.ones((1, 1), jnp.float32)
        tbl_ref[_ROW_1:_ROW_1 + 1, 60:61] = jnp.ones((1, 1), jnp.float32)
        w1p_ref[...] = jnp.zeros_like(w1p_ref)
        w1p_ref[0:60, 0:100] = w1_ref[...]
        w1p_ref[60:61, 0:100] = b1_ref[...][None, :]
        w1p_ref[60:61, 100:101] = jnp.ones((1, 1), jnp.float32)
        w2e_ref[...] = jnp.zeros_like(w2e_ref)
        w2e_ref[0:100, 0:10] = w2_ref[...]
        w2e_ref[100:101, 0:10] = b2_ref[...][None, :]
        w2e_ref[100:101, 10:11] = jnp.ones((1, 1), jnp.float32)
        w3e_ref[...] = jnp.zeros_like(w3e_ref)
        w3e_ref[0:10, 0:1] = w3_ref[...]
        w3e_ref[10:11, 0:1] = b3_ref[...][None, :]

    bblk = year_ref.shape[1]
    riota = lax.broadcasted_iota(jnp.int32, (128, bblk), 0)
    hot = ((riota == month_ref[...] + _OFF_M)
           | (riota == day_ref[...] + _OFF_D)
           | (riota == weekday_ref[...] + _OFF_W)
           | (riota == stores_ref[...] + _OFF_S)
           | (riota == items_ref[...] + _OFF_I)
           | (riota == _ROW_1))
    oh = jnp.where(riota == _ROW_Y, year_ref[...],
                   hot.astype(jnp.float32))
    hc = _dgT(tbl_ref[...], oh)
    h1 = jnp.maximum(_dgT(w1p_ref[...], hc), 0.0)
    h2 = jnp.maximum(_dgT(w2e_ref[...], h1), 0.0)
    out_ref[...] = _dgT(w3e_ref[...], h2)


def kernel(year, month, day, weekday, stores, items, emb_month, emb_day,
           emb_weekday, emb_stores, emb_items, W1, b1, W2, b2, W3, b3):
    B = year.shape[0]
    bblk = 4096
    grid = (B // bblk,)

    row = pl.BlockSpec((1, bblk), lambda i: (0, i))
    full = lambda s: pl.BlockSpec(s, lambda i: (0,) * len(s))
    out = pl.pallas_call(
        _fused_body,
        grid=grid,
        in_specs=[row] * 6 + [full(emb_month.shape), full(emb_day.shape),
                              full(emb_weekday.shape),
                              full(emb_stores.shape), full(emb_items.shape),
                              full(W1.shape), full(b1.shape), full(W2.shape),
                              full(b2.shape), full(W3.shape), full(b3.shape)],
        out_specs=row,
        out_shape=jax.ShapeDtypeStruct((1, B), jnp.float32),
        scratch_shapes=[pltpu.VMEM((128, 64), jnp.float32),
                        pltpu.VMEM((64, 104), jnp.float32),
                        pltpu.VMEM((104, 16), jnp.float32),
                        pltpu.VMEM((16, 1), jnp.float32)],
        compiler_params=pltpu.CompilerParams(
            dimension_semantics=("arbitrary",)),
    )(year.reshape(1, B), month.reshape(1, B), day.reshape(1, B),
      weekday.reshape(1, B), stores.reshape(1, B), items.reshape(1, B),
      emb_month, emb_day, emb_weekday, emb_stores, emb_items,
      W1, b1, W2, b2, W3, b3)
    return out.reshape(B, 1)


# speedup vs baseline: 25.6245x; 1.0639x over previous
import jax
import jax.numpy as jnp
from jax import lax
from jax.experimental import pallas as pl
from jax.experimental.pallas import tpu as pltpu

_OFF_M, _OFF_D, _OFF_W, _OFF_S, _OFF_I, _ROW_Y, _ROW_1 = (
    0, 13, 45, 53, 64, 115, 116)


def _dgT(a, b):
    return lax.dot_general(a, b, (((0,), (0,)), ((), ())),
                           preferred_element_type=jnp.float32)


def _fused_body(year_ref, month_ref, day_ref, weekday_ref, stores_ref,
                items_ref, emb_month_ref, emb_day_ref, emb_weekday_ref,
                emb_stores_ref, emb_items_ref, w1_ref, b1_ref, w2_ref,
                b2_ref, w3_ref, b3_ref, out_ref, tbl_ref, w1p_ref,
                w2e_ref, w3e_ref):
    i = pl.program_id(0)

    @pl.when(i == 0)
    def _assemble():
        tbl_ref[...] = jnp.zeros_like(tbl_ref)
        tbl_ref[_OFF_M:_OFF_M + 13, 1:8] = emb_month_ref[...]
        tbl_ref[_OFF_D:_OFF_D + 32, 8:24] = emb_day_ref[...]
        tbl_ref[_OFF_W:_OFF_W + 8, 24:28] = emb_weekday_ref[...]
        tbl_ref[_OFF_S:_OFF_S + 11, 28:34] = emb_stores_ref[...]
        tbl_ref[_OFF_I:_OFF_I + 51, 34:60] = emb_items_ref[...]
        tbl_ref[_ROW_Y:_ROW_Y + 1, 0:1] = jnp.ones((1, 1), jnp.float32)
        tbl_ref[_ROW_1:_ROW_1 + 1, 60:61] = jnp.ones((1, 1), jnp.float32)
        w1p_ref[...] = jnp.zeros_like(w1p_ref)
        w1p_ref[0:60, 0:100] = w1_ref[...]
        w1p_ref[60:61, 0:100] = b1_ref[...][None, :]
        w1p_ref[60:61, 100:101] = jnp.ones((1, 1), jnp.float32)
        w2e_ref[...] = jnp.zeros_like(w2e_ref)
        w2e_ref[0:100, 0:10] = w2_ref[...]
        w2e_ref[100:101, 0:10] = b2_ref[...][None, :]
        w2e_ref[100:101, 10:11] = jnp.ones((1, 1), jnp.float32)
        w3e_ref[...] = jnp.zeros_like(w3e_ref)
        w3e_ref[0:10, 0:1] = w3_ref[...]
        w3e_ref[10:11, 0:1] = b3_ref[...][None, :]

    bblk = year_ref.shape[1]
    riota = lax.broadcasted_iota(jnp.int32, (128, bblk), 0)
    hot = ((riota == month_ref[...] + _OFF_M)
           | (riota == day_ref[...] + _OFF_D)
           | (riota == weekday_ref[...] + _OFF_W)
           | (riota == stores_ref[...] + _OFF_S)
           | (riota == items_ref[...] + _OFF_I)
           | (riota == _ROW_1))
    oh = jnp.where(riota == _ROW_Y, year_ref[...],
                   hot.astype(jnp.float32))
    hc = _dgT(tbl_ref[...], oh)
    h1 = jnp.maximum(_dgT(w1p_ref[...], hc), 0.0)
    h2 = jnp.maximum(_dgT(w2e_ref[...], h1), 0.0)
    out_ref[...] = _dgT(w3e_ref[...], h2)


def kernel(year, month, day, weekday, stores, items, emb_month, emb_day,
           emb_weekday, emb_stores, emb_items, W1, b1, W2, b2, W3, b3):
    B = year.shape[0]
    bblk = 16384
    grid = (B // bblk,)

    row = pl.BlockSpec((1, bblk), lambda i: (0, i))
    full = lambda s: pl.BlockSpec(s, lambda i: (0,) * len(s))
    out = pl.pallas_call(
        _fused_body,
        grid=grid,
        in_specs=[row] * 6 + [full(emb_month.shape), full(emb_day.shape),
                              full(emb_weekday.shape),
                              full(emb_stores.shape), full(emb_items.shape),
                              full(W1.shape), full(b1.shape), full(W2.shape),
                              full(b2.shape), full(W3.shape), full(b3.shape)],
        out_specs=row,
        out_shape=jax.ShapeDtypeStruct((1, B), jnp.float32),
        scratch_shapes=[pltpu.VMEM((128, 64), jnp.float32),
                        pltpu.VMEM((64, 104), jnp.float32),
                        pltpu.VMEM((104, 16), jnp.float32),
                        pltpu.VMEM((16, 1), jnp.float32)],
        compiler_params=pltpu.CompilerParams(
            dimension_semantics=("arbitrary",)),
    )(year.reshape(1, B), month.reshape(1, B), day.reshape(1, B),
      weekday.reshape(1, B), stores.reshape(1, B), items.reshape(1, B),
      emb_month, emb_day, emb_weekday, emb_stores, emb_items,
      W1, b1, W2, b2, W3, b3)
    return out.reshape(B, 1)
